# baseline (speedup 1.0000x reference)
import functools

import jax
import jax.numpy as jnp
from jax.experimental import pallas as pl


N = 8192
F = 512
BM = 512


def _xw_kernel(x_ref, w_ref, o_ref):
    o_ref[...] = jnp.dot(x_ref[...], w_ref[...],
                         preferred_element_type=jnp.float32)


def _layer1_kernel(a_ref, u_ref, b1_ref, h_ref):
    t = jnp.dot(a_ref[...], u_ref[...], preferred_element_type=jnp.float32)
    h_ref[...] = jnp.maximum(t + b1_ref[0, 0], 0.0)


def _layer2_kernel(a_ref, h_ref, w2_ref, b2_ref, o_ref):
    t = jnp.dot(a_ref[...], h_ref[...], preferred_element_type=jnp.float32)
    o_ref[...] = jnp.maximum(t * w2_ref[...] + b2_ref[...], 0.0)


@jax.jit
def kernel(x, a, W1, b1, W2, b2):
    u = pl.pallas_call(
        _xw_kernel,
        grid=(N // 1024,),
        in_specs=[
            pl.BlockSpec((1024, F), lambda i: (i, 0)),
            pl.BlockSpec((F, 1), lambda i: (0, 0)),
        ],
        out_specs=pl.BlockSpec((1024, 1), lambda i: (i, 0)),
        out_shape=jax.ShapeDtypeStruct((N, 1), jnp.float32),
    )(x, W1)

    b1_2d = b1.reshape(1, 1)
    h = pl.pallas_call(
        _layer1_kernel,
        grid=(N // BM,),
        in_specs=[
            pl.BlockSpec((BM, N), lambda i: (i, 0)),
            pl.BlockSpec((N, 1), lambda i: (0, 0)),
            pl.BlockSpec((1, 1), lambda i: (0, 0)),
        ],
        out_specs=pl.BlockSpec((BM, 1), lambda i: (i, 0)),
        out_shape=jax.ShapeDtypeStruct((N, 1), jnp.float32),
    )(a, u, b1_2d)

    w2_2d = W2.reshape(1, 2)
    b2_2d = b2.reshape(1, 2)
    out = pl.pallas_call(
        _layer2_kernel,
        grid=(N // BM,),
        in_specs=[
            pl.BlockSpec((BM, N), lambda i: (i, 0)),
            pl.BlockSpec((N, 1), lambda i: (0, 0)),
            pl.BlockSpec((1, 2), lambda i: (0, 0)),
            pl.BlockSpec((1, 2), lambda i: (0, 0)),
        ],
        out_specs=pl.BlockSpec((BM, 2), lambda i: (i, 0)),
        out_shape=jax.ShapeDtypeStruct((N, 2), jnp.float32),
    )(a, h, w2_2d, b2_2d)

    return out


# speedup vs baseline: 1.0829x; 1.0829x over previous
import jax
import jax.numpy as jnp
from jax import lax
from jax.experimental import pallas as pl
from jax.experimental.pallas import tpu as pltpu


N = 8192
F = 512
BM = 512
NB = N // BM


def _gcn_kernel(a_ref, x_ref, w1_ref, b1_ref, w2_ref, b2_ref,
                o_ref, u_s, h_s):
    i = pl.program_id(0)

    @pl.when(i == 0)
    def _():
        u_s[...] = jnp.dot(x_ref[...], w1_ref[...],
                           preferred_element_type=jnp.float32)

    @pl.when(i < NB)
    def _():
        t = jnp.dot(a_ref[...], u_s[...],
                    preferred_element_type=jnp.float32)
        h_s[pl.ds(i * BM, BM), :] = jnp.maximum(t + b1_ref[0, 0], 0.0)

    @pl.when(i >= NB)
    def _():
        t = jnp.dot(a_ref[...], h_s[...],
                    preferred_element_type=jnp.float32)
        o_ref[...] = jnp.maximum(t * w2_ref[...] + b2_ref[...], 0.0)


@jax.jit
def kernel(x, a, W1, b1, W2, b2):
    b1_2d = b1.reshape(1, 1)
    w2_2d = W2.reshape(1, 2)
    b2_2d = b2.reshape(1, 2)
    return pl.pallas_call(
        _gcn_kernel,
        grid=(2 * NB,),
        in_specs=[
            pl.BlockSpec((BM, N), lambda i: (lax.rem(i, NB), 0)),
            pl.BlockSpec((N, F), lambda i: (0, 0)),
            pl.BlockSpec((F, 1), lambda i: (0, 0)),
            pl.BlockSpec((1, 1), lambda i: (0, 0)),
            pl.BlockSpec((1, 2), lambda i: (0, 0)),
            pl.BlockSpec((1, 2), lambda i: (0, 0)),
        ],
        out_specs=pl.BlockSpec((BM, 2), lambda i: (lax.max(i - NB, 0), 0)),
        out_shape=jax.ShapeDtypeStruct((N, 2), jnp.float32),
        scratch_shapes=[
            pltpu.VMEM((N, 1), jnp.float32),
            pltpu.VMEM((N, 1), jnp.float32),
        ],
        compiler_params=pltpu.CompilerParams(
            dimension_semantics=("arbitrary",),
        ),
    )(a, x, W1, b1_2d, w2_2d, b2_2d)
